# Initial kernel scaffold; baseline (speedup 1.0000x reference)
import jax
import jax.numpy as jnp
from jax.experimental import pallas as pl


def kernel(x, edge_index, edge_attr, params):
    raise NotImplementedError("write your pallas kernel here")


# speedup vs baseline: 1.8207x; 1.8207x over previous
import functools

import jax
import jax.numpy as jnp
from jax import lax
from jax.experimental import pallas as pl
from jax.experimental.pallas import tpu as pltpu
from jax.experimental.pallas import tpu_sc as plsc

N = 10000
E = 320000
NP = 10240
D = 16
DM = 8
BE = 2000
NB = E // BE
NW = 32
GC = (2 * E) // NW
GCH = 4000
SC_E = E // NW
NPT = NP // 16

def _mesh():
    return plsc.VectorSubcoreMesh(core_axis_name="c", subcore_axis_name="s")


_SC_PARAMS = pltpu.CompilerParams(use_tc_tiling_on_sc=False)


def _gather_body(table, idxall, out, idx_v, rows_v, sem):
    wid = lax.axis_index("s") * 2 + lax.axis_index("c")
    base = wid * GC
    for t in range(GC // GCH):
        off = base + t * GCH
        pltpu.sync_copy(idxall.at[pl.ds(off, GCH)], idx_v)
        pltpu.async_copy(table.at[idx_v], rows_v, sem).wait()
        pltpu.sync_copy(rows_v, out.at[pl.ds(off, GCH)])


@functools.partial(jax.jit)
def _sc_gather(table, idxall):
    k = pl.kernel(
        _gather_body,
        out_type=jax.ShapeDtypeStruct((2 * E, D), jnp.float32),
        mesh=_mesh(),
        scratch_types=[
            pltpu.VMEM((GCH,), jnp.int32),
            pltpu.VMEM((GCH, D), jnp.float32),
            pltpu.SemaphoreType.DMA,
        ],
        compiler_params=_SC_PARAMS,
    )
    return k(table, idxall)


def _scatter_body(m, dst, zer, out, idx_v, m_v, acc):
    c = lax.axis_index("c")
    s = lax.axis_index("s")
    wid = s * 2 + c
    pltpu.sync_copy(zer.at[pl.ds(s * NPT, NPT)], acc.at[pl.ds(s * NPT, NPT)])
    plsc.subcore_barrier()
    pltpu.sync_copy(dst.at[pl.ds(wid * SC_E, SC_E)], idx_v)
    pltpu.sync_copy(m.at[pl.ds(wid * SC_E, SC_E)], m_v)
    pltpu.sync_copy(m_v, acc.at[idx_v], add=True)
    plsc.subcore_barrier()
    pltpu.sync_copy(acc.at[pl.ds(s * NPT, NPT)], out.at[c].at[pl.ds(s * NPT, NPT)])


@functools.partial(jax.jit)
def _sc_scatter(m, dst, zer):
    k = pl.kernel(
        _scatter_body,
        out_type=jax.ShapeDtypeStruct((2, NP, DM), jnp.float32),
        mesh=_mesh(),
        scratch_types=[
            pltpu.VMEM((SC_E,), jnp.int32),
            pltpu.VMEM((SC_E, DM), jnp.float32),
            pltpu.VMEM_SHARED((NP, DM), jnp.float32),
        ],
        compiler_params=_SC_PARAMS,
    )
    return k(m, dst, zer)


def _dot3(a, b):
    ah = a.astype(jnp.bfloat16)
    al = (a - ah.astype(jnp.float32)).astype(jnp.bfloat16)
    bh = b.astype(jnp.bfloat16)
    bl = (b - bh.astype(jnp.float32)).astype(jnp.bfloat16)
    d = lambda u, v: jnp.dot(u, v, preferred_element_type=jnp.float32)
    return d(ah, bh) + (d(ah, bl) + d(al, bh))


def _dotp(a, b):
    return jnp.dot(a, b, preferred_element_type=jnp.float32)


def _full(a):
    nd = a.ndim
    return pl.BlockSpec(a.shape, lambda i, _n=nd: (0,) * _n)


def _enc_body(x, w, b, h):
    h[...] = _dotp(x[...], w[...]) + b[...]


BN = 2000
NNB = N // BN


def _rowspec(c):
    return pl.BlockSpec((BN, c), lambda i: (i, 0))


def _encoder(x, w, b):
    return pl.pallas_call(
        _enc_body,
        grid=(NNB,),
        in_specs=[_rowspec(128), _full(w), _full(b)],
        out_specs=_rowspec(D),
        out_shape=jax.ShapeDtypeStruct((N, D), jnp.float32),
    )(x, w, b)


def _edge_body(ce, gd, gs, e, w1, b1, w2, b2, w3, b3, m):
    zc = jnp.concatenate([gd[..., :7], gs[..., :7], e[..., :ce]], axis=1)
    z = jnp.maximum(_dotp(zc, w1[...]) + b1[...], 0.0)
    z = jnp.maximum(_dotp(z, w2[...]) + b2[...], 0.0)
    m[...] = _dotp(z, w3[...]) + b3[...]


def _edge_mlp(gall, e, ce, w1, b1, w2, b2, w3, b3):
    specs = [
        pl.BlockSpec((BE, D), lambda i: (i, 0)),
        pl.BlockSpec((BE, D), lambda i: (i + NB, 0)),
        pl.BlockSpec((BE, e.shape[1]), lambda i: (i, 0)),
    ] + [_full(a) for a in (w1, b1, w2, b2, w3, b3)]
    return pl.pallas_call(
        functools.partial(_edge_body, ce),
        grid=(NB,),
        in_specs=specs,
        out_specs=pl.BlockSpec((BE, DM), lambda i: (i, 0)),
        out_shape=jax.ShapeDtypeStruct((E, DM), jnp.float32),
    )(gall, gall, e, w1, b1, w2, b2, w3, b3)


def _edge_c1_body(gd, gs, ea, e1, e2, e3,
                  v1, c1, u2, c2, u3, c3,
                  w1, b1, w2, b2, w3, b3,
                  m, wout):
    tc = jnp.concatenate([ea[...], e1[..., :4], e2[..., :4], e3[..., :4]], axis=1)
    t = jnp.maximum(_dotp(tc, v1[...]) + c1[...], 0.0)
    t = jnp.maximum(_dotp(t, u2[...]) + c2[...], 0.0)
    w = jax.nn.sigmoid(_dotp(t, u3[...]) + c3[...])
    wout[...] = w
    zc = jnp.concatenate([gd[..., :7], gs[..., :7], w, tc], axis=1)
    z = jnp.maximum(_dotp(zc, w1[...]) + b1[...], 0.0)
    z = jnp.maximum(_dotp(z, w2[...]) + b2[...], 0.0)
    m[...] = _dotp(z, w3[...]) + b3[...]


def _edge_c1(gall, ea, e1, e2, e3, wargs, relargs):
    specs = [
        pl.BlockSpec((BE, D), lambda i: (i, 0)),
        pl.BlockSpec((BE, D), lambda i: (i + NB, 0)),
        pl.BlockSpec((BE, 4), lambda i: (i, 0)),
        pl.BlockSpec((BE, DM), lambda i: (i, 0)),
        pl.BlockSpec((BE, DM), lambda i: (i, 0)),
        pl.BlockSpec((BE, DM), lambda i: (i, 0)),
    ] + [_full(a) for a in wargs] + [_full(a) for a in relargs]
    return pl.pallas_call(
        _edge_c1_body,
        grid=(NB,),
        in_specs=specs,
        out_specs=(
            pl.BlockSpec((BE, DM), lambda i: (i, 0)),
            pl.BlockSpec((BE, 1), lambda i: (i, 0)),
        ),
        out_shape=(
            jax.ShapeDtypeStruct((E, DM), jnp.float32),
            jax.ShapeDtypeStruct((E, 1), jnp.float32),
        ),
    )(gall, gall, ea, e1, e2, e3, *wargs, *relargs)


def _node_body(ar, h, xl, a0, a1, w1, b1, w2, b2, w3, b3, out):
    zc = jnp.concatenate([xl[..., :7], (a0[...] + a1[...])[:, :ar]], axis=1)
    z = jnp.maximum(_dotp(zc, w1[...]) + b1[...], 0.0)
    z = jnp.maximum(_dotp(z, w2[...]) + b2[...], 0.0)
    out[...] = h[...] + (_dotp(z, w3[...]) + b3[...])


def _node_mlp(h, xl, a0, a1, ar, w1, b1, w2, b2, w3, b3):
    ws = (w1, b1, w2, b2, w3, b3)
    return pl.pallas_call(
        functools.partial(_node_body, ar),
        grid=(NNB,),
        in_specs=[_rowspec(D), _rowspec(D), _rowspec(DM), _rowspec(DM)]
        + [_full(a) for a in ws],
        out_specs=_rowspec(D),
        out_shape=jax.ShapeDtypeStruct((N, D), jnp.float32),
    )(h, xl, a0, a1, *ws)


def _final_body(h, xl, a0, a1, w1, b1, w2, b2, w3, b3,
                wb1, cb1, wb2, cb2, wb3, cb3,
                wx1, cx1, wx2, cx2, wx3, cx3, beta, hc):
    zc = jnp.concatenate([xl[..., :7], a0[...] + a1[...]], axis=1)
    z = jnp.maximum(_dotp(zc, w1[...]) + b1[...], 0.0)
    z = jnp.maximum(_dotp(z, w2[...]) + b2[...], 0.0)
    hc3 = h[..., :7] + (_dotp(z, w3[...]) + b3[...])
    t = jnp.maximum(_dotp(hc3, wb1[...]) + cb1[...], 0.0)
    t = jnp.maximum(_dotp(t, wb2[...]) + cb2[...], 0.0)
    beta[...] = jax.nn.sigmoid(_dotp(t, wb3[...]) + cb3[...])
    t = jnp.maximum(_dotp(hc3, wx1[...]) + cx1[...], 0.0)
    t = jnp.maximum(_dotp(t, wx2[...]) + cx2[...], 0.0)
    hc[...] = _dotp(t, wx3[...]) + cx3[...]


def _final_mlp(h, xl, a0, a1, objargs, bargs, xargs):
    ws = objargs + bargs + xargs
    return pl.pallas_call(
        _final_body,
        grid=(NNB,),
        in_specs=[_rowspec(D), _rowspec(D), _rowspec(DM), _rowspec(DM)]
        + [_full(a) for a in ws],
        out_specs=(_rowspec(1), _rowspec(3)),
        out_shape=(
            jax.ShapeDtypeStruct((N, 1), jnp.float32),
            jax.ShapeDtypeStruct((N, 3), jnp.float32),
        ),
    )(h, xl, a0, a1, *ws)


def _padr(w, rows):
    return jnp.pad(w, ((0, rows - w.shape[0]), (0, 0)))


def _padc(w, cols):
    return jnp.pad(w, ((0, 0), (0, cols - w.shape[1])))


def _rel_args(p):
    return (p['W1'], p['b1'][None, :], p['W2'], p['b2'][None, :],
            _padc(p['W3'], DM), _padc(p['b3'][None, :], DM))


def _obj_args(p):
    return (p['W1'], p['b1'][None, :], p['W2'], p['b2'][None, :],
            _padc(p['W3'], D), _padc(p['b3'][None, :], D))


def kernel(x, edge_index, edge_attr, params):
    p = params
    dst = edge_index[1]
    src = edge_index[0]
    idx_all = jnp.concatenate([dst, src], axis=0)
    zer = jnp.zeros((NP, DM), jnp.float32)

    h = _encoder(x, _padc(p['encoder_W'], D), _padc(p['encoder_b'][None, :], D))

    def seg(m):
        s = jax.ops.segment_sum(m, dst, num_segments=N)
        return s, jnp.zeros_like(s)

    def layer(pl_, xl, e, ce, ar):
        gall = _sc_gather(xl, idx_all)
        m = _edge_mlp(gall, e, ce, *_rel_args(pl_['rel']))
        a0, a1 = seg(m)
        xn = _node_mlp(h, xl, a0, a1, ar, *_obj_args(pl_['obj']))
        return xn, m

    x1, e1 = layer(p['in_w1'], h, edge_attr, 4, 4)
    x2, e2 = layer(p['in_w2'], x1, e1, 4, 4)
    x3, e3 = layer(p['in_w3'], x2, e2, 4, 4)

    pw = p['W']
    wargs = (pw['W1'], pw['b1'][None, :], pw['W2'],
             pw['b2'][None, :], pw['W3'], pw['b3'][None, :])
    pc = p['in_c1']['rel']
    relargs = (pc['W1'], pc['b1'][None, :], pc['W2'],
               pc['b2'][None, :], _padc(pc['W3'], DM), _padc(pc['b3'][None, :], DM))
    gall = _sc_gather(x3, idx_all)
    ec1, edge_weights = _edge_c1(gall, edge_attr, e1, e2, e3, wargs, relargs)
    a0, a1 = seg(ec1)
    x4 = _node_mlp(h, x3, a0, a1, 8, *_obj_args(p['in_c1']['obj']))

    x5, ec2 = layer(p['in_c2'], x4, ec1, 8, 8)

    gall = _sc_gather(x5, idx_all)
    ec3 = _edge_mlp(gall, ec2, 8, *_rel_args(p['in_c3']['rel']))
    agg = _sc_scatter(ec3, dst, zer)
    pb, px, po = p['B'], p['X'], p['in_c3']['obj']
    oa = (po['W1'], po['b1'][None, :], po['W2'], po['b2'][None, :],
          po['W3'], po['b3'][None, :])
    bargs = (pb['W1'], pb['b1'][None, :], pb['W2'], pb['b2'][None, :],
             pb['W3'], pb['b3'][None, :])
    xargs = (px['W1'], px['b1'][None, :], px['W2'], px['b2'][None, :],
             px['W3'], px['b3'][None, :])
    beta, hc = _final_mlp(h, x5, agg[0, :N], agg[1, :N], oa, bargs, xargs)

    return (edge_weights, hc, beta)
